# baseline (speedup 1.0000x reference)
import functools

import jax
import jax.numpy as jnp
from jax.experimental import pallas as pl

N, E, D, G, C = 10000, 320000, 128, 64, 10
RATIO = 0.5
EPS = 1e-5


def _combine_kernel(p0, p1, p2, lw0, lw1, lw2, lb, out_ref):
    acc = jnp.dot(p0[...], lw0[...], preferred_element_type=jnp.float32)
    acc += jnp.dot(p1[...], lw1[...], preferred_element_type=jnp.float32)
    acc += jnp.dot(p2[...], lw2[...], preferred_element_type=jnp.float32)
    out_ref[...] = acc + lb[...]


def _gcn_conv(h, W, b, src, dst, e_act, n_act):
    hw = h @ W
    deg = jnp.zeros((h.shape[0],), h.dtype).at[dst].add(e_act) + n_act
    dinv = jnp.where(deg > 0, 1.0 / jnp.sqrt(jnp.where(deg > 0, deg, 1.0)), 0.0)
    coef = dinv[src] * dinv[dst] * e_act
    out = jnp.zeros_like(hw).at[dst].add(hw[src] * coef[:, None])
    out = out + hw * (dinv * dinv * n_act)[:, None]
    return (out + b) * n_act[:, None]


def _topk_mask(score, batch, n_act_bool):
    n_act = n_act_bool.astype(jnp.float32)
    masked = jnp.where(n_act_bool, score, -jnp.inf)
    order = jnp.lexsort((-masked, batch))
    counts_all = jnp.bincount(batch, length=G)
    offsets = jnp.concatenate([jnp.zeros((1,), counts_all.dtype), jnp.cumsum(counts_all)[:-1]])
    Nn = score.shape[0]
    ranks = jnp.zeros((Nn,), jnp.int32).at[order].set(
        (jnp.arange(Nn) - offsets[batch[order]]).astype(jnp.int32))
    n_per = jnp.zeros((G,), jnp.float32).at[batch].add(n_act)
    k = jnp.ceil(RATIO * n_per).astype(jnp.int32)
    return n_act_bool & (ranks < k[batch])


def kernel(x, edge_index, batch, W0, b0, Ws0, bs0, g0, be0, W1, b1, Ws1, bs1, g1, be1, LW0, Lb0, LW1, Lb1, LW2, Lb2):
    src, dst = edge_index[0], edge_index[1]
    n_act = jnp.ones((x.shape[0],), bool)
    e_act = jnp.ones((src.shape[0],), jnp.float32)
    convW = [(W0, b0), (W1, b1)]
    scoreW = [(Ws0, bs0), (Ws1, bs1)]
    bnW = [(g0, be0), (g1, be1)]
    hidden = [x]
    h = x
    for l in range(2):
        Wc, bc = convW[l]
        h = _gcn_conv(h, Wc, bc, src, dst, e_act, n_act.astype(jnp.float32))
        Wsc, bsc = scoreW[l]
        score = _gcn_conv(h, Wsc, bsc, src, dst, e_act, n_act.astype(jnp.float32))[:, 0]
        keep = _topk_mask(score, batch, n_act)
        m = keep.astype(jnp.float32)
        h = h * jnp.tanh(score)[:, None] * m[:, None]
        n_act = keep
        e_act = e_act * m[src] * m[dst]
        cnt = jnp.maximum(m.sum(), 1.0)
        mean = (h * m[:, None]).sum(0) / cnt
        var = (((h - mean) ** 2) * m[:, None]).sum(0) / cnt
        g, be = bnW[l]
        h = ((h - mean) / jnp.sqrt(var + EPS) * g + be) * m[:, None]
        h = jax.nn.relu(h)
        hidden.append(h)
    pooled = [jax.ops.segment_sum(hh, batch, num_segments=G) for hh in hidden]
    lb = Lb0 + Lb1 + Lb2
    out = pl.pallas_call(
        _combine_kernel,
        out_shape=jax.ShapeDtypeStruct((G, C), jnp.float32),
    )(pooled[0], pooled[1], pooled[2], LW0, LW1, LW2, lb[None, :])
    return out


# speedup vs baseline: 18.5682x; 18.5682x over previous
import functools
from typing import Any

import jax
import jax.numpy as jnp
from jax import lax
from jax.experimental import pallas as pl
from jax.experimental.pallas import tpu as pltpu
from jax.experimental.pallas import tpu_sc as plsc

RATIO = 0.5
EPS = 1e-5

NW = 32
CHUNK = 128
ZCH = 128


def _round_up(x, m):
    return (x + m - 1) // m * m


def _sc_scatter_body(n_pad, w, n_ch, u_hbm, src_hbm, dst_hbm, out_hbm,
                     sidx_v, didx_v, rows_v, acc_sp, sem):
    c = lax.axis_index("c")
    s = lax.axis_index("s")
    wid = s * 2 + c
    stripe = n_pad // 16
    pltpu.sync_copy(src_hbm.at[wid], sidx_v)
    pltpu.sync_copy(dst_hbm.at[wid], didx_v)
    pltpu.sync_copy(u_hbm.at[pl.ds(n_pad - ZCH, ZCH)], rows_v.at[pl.ds(0, ZCH)])
    for z in range(stripe // ZCH):
        pltpu.sync_copy(rows_v.at[pl.ds(0, ZCH)],
                        acc_sp.at[pl.ds(s * stripe + z * ZCH, ZCH)])
    plsc.subcore_barrier()

    def chunk(j, carry):
        pltpu.async_copy(u_hbm.at[sidx_v.at[j]], rows_v, sem).wait()
        pltpu.sync_copy(rows_v, acc_sp.at[didx_v.at[j]], add=True)
        return carry
    lax.fori_loop(0, n_ch, chunk, 0)
    plsc.subcore_barrier()
    pltpu.sync_copy(acc_sp.at[pl.ds(s * stripe, stripe)],
                    out_hbm.at[c, pl.ds(s * stripe, stripe)])


def _make_sc_scatter(n_pad, w, n_ch):
    mesh = plsc.VectorSubcoreMesh(core_axis_name="c", subcore_axis_name="s")
    return pl.kernel(
        functools.partial(_sc_scatter_body, n_pad, w, n_ch),
        out_type=jax.ShapeDtypeStruct((2, n_pad, w), jnp.float32),
        mesh=mesh,
        compiler_params=pltpu.CompilerParams(use_tc_tiling_on_sc=False),
        scratch_types=[
            pltpu.VMEM((n_ch, CHUNK), jnp.int32),
            pltpu.VMEM((n_ch, CHUNK), jnp.int32),
            pltpu.VMEM((CHUNK, w), jnp.float32),
            pltpu.VMEM_SHARED((n_pad, w), jnp.float32),
            pltpu.SemaphoreType.DMA,
        ],
    )


def _mm_body(a_ref, w_ref, o_ref):
    o_ref[...] = jnp.dot(a_ref[...], w_ref[...], preferred_element_type=jnp.float32)


def _matmul(a, w, bn=512):
    n, k = a.shape
    m = w.shape[1]
    return pl.pallas_call(
        _mm_body,
        grid=(n // bn,),
        in_specs=[pl.BlockSpec((bn, k), lambda i: (i, 0)),
                  pl.BlockSpec((k, m), lambda i: (0, 0))],
        out_specs=pl.BlockSpec((bn, m), lambda i: (i, 0)),
        out_shape=jax.ShapeDtypeStruct((n, m), jnp.float32),
    )(a, w)


def _scale_body(hw_ref, c0_ref, c1_ref, act_ref, u_ref, a_ref):
    cnt = c0_ref[:, :1] + c1_ref[:, :1]
    act = act_ref[...]
    a = act / jnp.sqrt(cnt + 1.0)
    a_ref[...] = a
    u_ref[...] = hw_ref[...] * a


def _scale_table(hw_ext, cnt0, cnt1, act, bn=512):
    n, w = hw_ext.shape
    return pl.pallas_call(
        _scale_body,
        grid=(n // bn,),
        in_specs=[pl.BlockSpec((bn, w), lambda i: (i, 0)),
                  pl.BlockSpec((bn, 16), lambda i: (i, 0)),
                  pl.BlockSpec((bn, 16), lambda i: (i, 0)),
                  pl.BlockSpec((bn, 1), lambda i: (i, 0))],
        out_specs=[pl.BlockSpec((bn, w), lambda i: (i, 0)),
                   pl.BlockSpec((bn, 1), lambda i: (i, 0))],
        out_shape=[jax.ShapeDtypeStruct((n, w), jnp.float32),
                   jax.ShapeDtypeStruct((n, 1), jnp.float32)],
    )(hw_ext, cnt0, cnt1, act)


def _post_body(p0_ref, p1_ref, hw_ref, a_ref, act_ref, bias_ref, ws_ref,
               h_ref, sw_ref, us_ref):
    a = a_ref[...]
    conv = a * (p0_ref[...] + p1_ref[...]) + hw_ref[...] * (a * a)
    res = (conv + bias_ref[...]) * act_ref[...]
    h_ref[...] = res
    sw = jnp.dot(res, ws_ref[...], preferred_element_type=jnp.float32)
    sw_ref[...] = sw
    us_ref[...] = jnp.broadcast_to(sw * a, us_ref.shape)


def _post_conv(p0, p1, hw, a, act, bias, ws_row, bn=512):
    n, d = hw.shape
    return pl.pallas_call(
        _post_body,
        grid=(n // bn,),
        in_specs=[pl.BlockSpec((bn, d), lambda i: (i, 0)),
                  pl.BlockSpec((bn, d), lambda i: (i, 0)),
                  pl.BlockSpec((bn, d), lambda i: (i, 0)),
                  pl.BlockSpec((bn, 1), lambda i: (i, 0)),
                  pl.BlockSpec((bn, 1), lambda i: (i, 0)),
                  pl.BlockSpec((1, d), lambda i: (0, 0)),
                  pl.BlockSpec((d, 1), lambda i: (0, 0))],
        out_specs=[pl.BlockSpec((bn, d), lambda i: (i, 0)),
                   pl.BlockSpec((bn, 1), lambda i: (i, 0)),
                   pl.BlockSpec((bn, 16), lambda i: (i, 0))],
        out_shape=[jax.ShapeDtypeStruct((n, d), jnp.float32),
                   jax.ShapeDtypeStruct((n, 1), jnp.float32),
                   jax.ShapeDtypeStruct((n, 16), jnp.float32)],
    )(p0, p1, hw, a, act, bias, ws_row)


def _score_body(q0_ref, q1_ref, sw_ref, a_ref, act_ref, bs_ref, s_ref):
    a = a_ref[...]
    accs = q0_ref[:, :1] + q1_ref[:, :1]
    s_ref[...] = (a * accs + sw_ref[...] * (a * a) + bs_ref[...]) * act_ref[...]


def _score_post(q0, q1, sw, a, act, bs, bn=512):
    n = sw.shape[0]
    return pl.pallas_call(
        _score_body,
        grid=(n // bn,),
        in_specs=[pl.BlockSpec((bn, 16), lambda i: (i, 0)),
                  pl.BlockSpec((bn, 16), lambda i: (i, 0)),
                  pl.BlockSpec((bn, 1), lambda i: (i, 0)),
                  pl.BlockSpec((bn, 1), lambda i: (i, 0)),
                  pl.BlockSpec((bn, 1), lambda i: (i, 0)),
                  pl.BlockSpec((1, 1), lambda i: (0, 0))],
        out_specs=pl.BlockSpec((bn, 1), lambda i: (i, 0)),
        out_shape=jax.ShapeDtypeStruct((n, 1), jnp.float32),
    )(q0, q1, sw, a, act, bs)


def _topk_body(jch, jblk, sc_ref, bc_ref, ac_ref, sr_ref, br_ref, ar_ref,
               ilo_ref, ihi_ref, jlo_ref, jhi_ref, keep_ref):
    p = pl.program_id(0)
    bi = sc_ref.shape[0]
    si = sc_ref[...]
    gi = bc_ref[...]
    ai = ac_ref[...]
    ii = (lax.broadcasted_iota(jnp.int32, (bi, 1), 0).astype(jnp.float32)
          + jnp.float32(bi) * p.astype(jnp.float32))
    my_lo = ilo_ref[p]
    my_hi = ihi_ref[p]

    def jstep(j, carry):
        rank, cnt = carry

        def live(_):
            sj = sr_ref[pl.ds(j, 1), :]
            gj = br_ref[pl.ds(j, 1), :]
            aj = ar_ref[pl.ds(j, 1), :]
            jj = (lax.broadcasted_iota(jnp.int32, (1, jblk), 1).astype(jnp.float32)
                  + jnp.float32(jblk) * j.astype(jnp.float32))
            same = (gi == gj) & (aj > 0.0)
            better = (sj > si) | ((sj == si) & (jj < ii))
            r = jnp.sum(jnp.where(same & better, 1.0, 0.0), axis=1, keepdims=True)
            c = jnp.sum(jnp.where(same, 1.0, 0.0), axis=1, keepdims=True)
            return rank + r, cnt + c

        ov = (my_lo <= jhi_ref[j]) & (jlo_ref[j] <= my_hi)
        return lax.cond(ov, live, lambda _: (rank, cnt), 0)

    rank, cnt = lax.fori_loop(0, jch, jstep, (jnp.zeros((bi, 1), jnp.float32),
                                              jnp.zeros((bi, 1), jnp.float32)))
    k = jnp.ceil(RATIO * cnt)
    keep = jnp.where((ai > 0.0) & (rank < k), 1.0, 0.0)
    keep_ref[...] = jnp.broadcast_to(keep, keep_ref.shape)


def _topk(score_c, batch_c, act_c, score_r, batch_r, act_r, ilo, ihi, jlo, jhi,
          bi=128, jblk=512):
    n = score_c.shape[0]
    jch = n // jblk
    return pl.pallas_call(
        functools.partial(_topk_body, jch, jblk),
        grid=(n // bi,),
        in_specs=[pl.BlockSpec((bi, 1), lambda i: (i, 0)),
                  pl.BlockSpec((bi, 1), lambda i: (i, 0)),
                  pl.BlockSpec((bi, 1), lambda i: (i, 0)),
                  pl.BlockSpec((jch, jblk), lambda i: (0, 0)),
                  pl.BlockSpec((jch, jblk), lambda i: (0, 0)),
                  pl.BlockSpec((jch, jblk), lambda i: (0, 0)),
                  pl.BlockSpec(memory_space=pltpu.SMEM),
                  pl.BlockSpec(memory_space=pltpu.SMEM),
                  pl.BlockSpec(memory_space=pltpu.SMEM),
                  pl.BlockSpec(memory_space=pltpu.SMEM)],
        out_specs=pl.BlockSpec((bi, 16), lambda i: (i, 0)),
        out_shape=jax.ShapeDtypeStruct((n, 16), jnp.float32),
    )(score_c, batch_c, act_c, score_r, batch_r, act_r, ilo, ihi, jlo, jhi)


def _stats_body(h_ref, s_ref, m_ref, sum_ref, sq_ref, c_ref):
    @pl.when(pl.program_id(0) == 0)
    def _init():
        sum_ref[...] = jnp.zeros_like(sum_ref)
        sq_ref[...] = jnp.zeros_like(sq_ref)
        c_ref[...] = jnp.zeros_like(c_ref)
    m = m_ref[...]
    hm = h_ref[...] * jnp.tanh(s_ref[...]) * m
    sum_ref[...] += jnp.sum(hm, axis=0, keepdims=True)
    sq_ref[...] += jnp.sum(hm * hm, axis=0, keepdims=True)
    c_ref[...] += jnp.sum(m, axis=0, keepdims=True)


def _bn_stats(h, score, m, bn=512):
    n, d = h.shape
    return pl.pallas_call(
        _stats_body,
        grid=(n // bn,),
        in_specs=[pl.BlockSpec((bn, d), lambda i: (i, 0)),
                  pl.BlockSpec((bn, 1), lambda i: (i, 0)),
                  pl.BlockSpec((bn, 1), lambda i: (i, 0))],
        out_specs=[pl.BlockSpec((1, d), lambda i: (0, 0)),
                   pl.BlockSpec((1, d), lambda i: (0, 0)),
                   pl.BlockSpec((1, 1), lambda i: (0, 0))],
        out_shape=[jax.ShapeDtypeStruct((1, d), jnp.float32),
                   jax.ShapeDtypeStruct((1, d), jnp.float32),
                   jax.ShapeDtypeStruct((1, 1), jnp.float32)],
    )(h, score, m)


def _bnorm_body(with_mm, h_ref, s_ref, m_ref, sum_ref, sq_ref, c_ref,
                g_ref, be_ref, w_ref, h_out, hw_out):
    cnt = jnp.maximum(c_ref[0, 0], 1.0)
    mean = sum_ref[...] / cnt
    var = sq_ref[...] / cnt - mean * mean
    inv = lax.rsqrt(var + EPS)
    m = m_ref[...]
    hm = h_ref[...] * jnp.tanh(s_ref[...]) * m
    hn = ((hm - mean) * inv * g_ref[...] + be_ref[...]) * m
    hn = jnp.maximum(hn, 0.0)
    h_out[...] = hn
    if with_mm:
        hw_out[...] = jnp.dot(hn, w_ref[...], preferred_element_type=jnp.float32)


def _bn_apply(h, score, m, ssum, ssq, scnt, g, be, wcat=None, bn=512):
    n, d = h.shape
    with_mm = wcat is not None
    in_specs = [pl.BlockSpec((bn, d), lambda i: (i, 0)),
                pl.BlockSpec((bn, 1), lambda i: (i, 0)),
                pl.BlockSpec((bn, 1), lambda i: (i, 0)),
                pl.BlockSpec((1, d), lambda i: (0, 0)),
                pl.BlockSpec((1, d), lambda i: (0, 0)),
                pl.BlockSpec((1, 1), lambda i: (0, 0)),
                pl.BlockSpec((1, d), lambda i: (0, 0)),
                pl.BlockSpec((1, d), lambda i: (0, 0))]
    out_specs = [pl.BlockSpec((bn, d), lambda i: (i, 0))]
    out_shape = [jax.ShapeDtypeStruct((n, d), jnp.float32)]
    args = [h, score, m, ssum, ssq, scnt, g[None, :], be[None, :]]
    if with_mm:
        w = wcat.shape[1]
        in_specs.append(pl.BlockSpec((d, w), lambda i: (0, 0)))
        out_specs.append(pl.BlockSpec((bn, w), lambda i: (i, 0)))
        out_shape.append(jax.ShapeDtypeStruct((n, w), jnp.float32))
        args.append(wcat)
    else:
        in_specs.append(pl.BlockSpec((d, 8), lambda i: (0, 0)))
        args.append(jnp.zeros((d, 8), jnp.float32))
    body = functools.partial(_bnorm_body, with_mm)
    if not with_mm:
        def body(h_ref, s_ref, m_ref, sum_ref, sq_ref, c_ref, g_ref, be_ref,
                 w_ref, h_out):
            _bnorm_body(False, h_ref, s_ref, m_ref, sum_ref, sq_ref, c_ref,
                        g_ref, be_ref, w_ref, h_out, None)
    res = pl.pallas_call(
        body,
        grid=(n // bn,),
        in_specs=in_specs,
        out_specs=out_specs,
        out_shape=out_shape,
    )(*args)
    return res if with_mm else (res[0], None)


def _combine_body(ng, x_ref, h1_ref, h2_ref, b_ref, lw0_ref, lw1_ref, lw2_ref,
                  lb_ref, o_ref, p0_ref, p1_ref, p2_ref):
    @pl.when(pl.program_id(0) == 0)
    def _init():
        p0_ref[...] = jnp.zeros_like(p0_ref)
        p1_ref[...] = jnp.zeros_like(p1_ref)
        p2_ref[...] = jnp.zeros_like(p2_ref)
    hp = jax.lax.Precision.HIGHEST
    gc = lax.broadcasted_iota(jnp.int32, (ng, 1), 0).astype(jnp.float32)
    pt = jnp.where(b_ref[0] == gc, 1.0, 0.0)
    p0_ref[...] += jnp.dot(pt, x_ref[...], preferred_element_type=jnp.float32,
                           precision=hp)
    p1_ref[...] += jnp.dot(pt, h1_ref[...], preferred_element_type=jnp.float32,
                           precision=hp)
    p2_ref[...] += jnp.dot(pt, h2_ref[...], preferred_element_type=jnp.float32,
                           precision=hp)

    @pl.when(pl.program_id(0) == pl.num_programs(0) - 1)
    def _final():
        out = jnp.dot(p0_ref[...], lw0_ref[...],
                      preferred_element_type=jnp.float32)
        out += jnp.dot(p1_ref[...], lw1_ref[...],
                       preferred_element_type=jnp.float32)
        out += jnp.dot(p2_ref[...], lw2_ref[...],
                       preferred_element_type=jnp.float32)
        o_ref[...] = out + lb_ref[...]


def _combine(x, h1, h2, batch_row, lw0, lw1, lw2, lbsum, ng, nc, bn=512):
    n, d = x.shape
    return pl.pallas_call(
        functools.partial(_combine_body, ng),
        grid=(n // bn,),
        in_specs=[pl.BlockSpec((bn, d), lambda i: (i, 0)),
                  pl.BlockSpec((bn, d), lambda i: (i, 0)),
                  pl.BlockSpec((bn, d), lambda i: (i, 0)),
                  pl.BlockSpec((1, 1, bn), lambda i: (i, 0, 0)),
                  pl.BlockSpec((d, nc), lambda i: (0, 0)),
                  pl.BlockSpec((d, nc), lambda i: (0, 0)),
                  pl.BlockSpec((d, nc), lambda i: (0, 0)),
                  pl.BlockSpec((1, nc), lambda i: (0, 0))],
        out_specs=pl.BlockSpec((ng, nc), lambda i: (0, 0)),
        out_shape=jax.ShapeDtypeStruct((ng, nc), jnp.float32),
        scratch_shapes=[pltpu.VMEM((ng, d), jnp.float32)] * 3,
    )(x, h1, h2, batch_row.reshape(n // bn, 1, bn), lw0, lw1, lw2, lbsum)


def kernel(x, edge_index, batch, W0, b0, Ws0, bs0, g0, be0, W1, b1, Ws1, bs1,
           g1, be1, LW0, Lb0, LW1, Lb1, LW2, Lb2):
    n, d = x.shape
    e = edge_index.shape[1]
    nc = LW0.shape[1]
    ng = 64
    n_pad = _round_up(n + ZCH, 2048)
    e_pad = _round_up(e, NW * CHUNK)
    n_ch = e_pad // (NW * CHUNK)
    n_trash = n_pad - n

    xp = jnp.pad(x, ((0, n_pad - n), (0, 0)))
    batch_p = jnp.pad(batch, (0, n_pad - n), constant_values=ng)
    bf = batch_p.astype(jnp.float32)
    batch_c = bf[:, None]
    batch_r = bf.reshape(-1, 512)
    bi_lo = batch_p.reshape(-1, 128)[:, 0]
    bi_hi = batch_p.reshape(-1, 128)[:, 127]
    jc_lo = batch_p.reshape(-1, 512)[:, 0]
    jc_hi = batch_p.reshape(-1, 512)[:, 511]

    npad_e = e_pad - e
    trash = n + (jnp.arange(npad_e, dtype=jnp.int32) % n_trash)
    src3 = jnp.concatenate([edge_index[0], trash]).reshape(NW, n_ch, CHUNK)
    dst3 = jnp.concatenate([edge_index[1], trash]).reshape(NW, n_ch, CHUNK)

    act = jnp.pad(jnp.ones((n, 1), jnp.float32), ((0, n_pad - n), (0, 0)))
    act_r = act.reshape(-1, 512)
    m_tbl = jnp.pad(jnp.ones((n, 16), jnp.float32), ((0, n_pad - n), (0, 0)))

    sc_cnt = _make_sc_scatter(n_pad, 16, n_ch)
    sc_conv = _make_sc_scatter(n_pad, d, n_ch)

    convW = [(b0[None, :], Ws0.reshape(d, 1), bs0.reshape(1, 1)),
             (b1[None, :], Ws1.reshape(d, 1), bs1.reshape(1, 1))]
    bnW = [(g0, be0), (g1, be1)]
    hidden = []
    hw = _matmul(xp, W0)
    for l in range(2):
        bias, ws_row, bs = convW[l]
        cntp = sc_cnt(m_tbl, src3, dst3)
        u, a = _scale_table(hw, cntp[0], cntp[1], act)
        accp = sc_conv(u, src3, dst3)
        hcv, sw, us16 = _post_conv(accp[0], accp[1], hw, a, act, bias, ws_row)
        qp = sc_cnt(us16, src3, dst3)
        score = _score_post(qp[0], qp[1], sw, a, act, bs)
        sr = score.reshape(-1, 512)
        keep16 = _topk(score, batch_c, act, sr, batch_r, act_r,
                       bi_lo, bi_hi, jc_lo, jc_hi)
        m = keep16[:, :1]
        ssum, ssq, scnt = _bn_stats(hcv, score, m)
        g, be = bnW[l]
        if l == 0:
            h, hw = _bn_apply(hcv, score, m, ssum, ssq, scnt, g, be, W1)
        else:
            h, _ = _bn_apply(hcv, score, m, ssum, ssq, scnt, g, be, None)
        hidden.append(h)
        act = m
        act_r = act.reshape(-1, 512)
        m_tbl = keep16

    lbsum = (Lb0 + Lb1 + Lb2)[None, :]
    out = _combine(xp, hidden[0], hidden[1], batch_r, LW0, LW1, LW2, lbsum,
                   ng, nc)
    return out


# speedup vs baseline: 19.2348x; 1.0359x over previous
import functools
from typing import Any

import jax
import jax.numpy as jnp
from jax import lax
from jax.experimental import pallas as pl
from jax.experimental.pallas import tpu as pltpu
from jax.experimental.pallas import tpu_sc as plsc

RATIO = 0.5
EPS = 1e-5

NW = 32
CHUNK = 96
ZCH = 64


def _round_up(x, m):
    return (x + m - 1) // m * m


def _sc_scatter_body(n_pad, w, n_ch, u_hbm, src_hbm, dst_hbm, out_hbm,
                     sidx_v, didx_v, rows_v, rows_b, acc_sp, semA, semB):
    c = lax.axis_index("c")
    s = lax.axis_index("s")
    wid = s * 2 + c
    stripe = n_pad // 16
    pltpu.sync_copy(src_hbm.at[wid], sidx_v)
    pltpu.sync_copy(dst_hbm.at[wid], didx_v)
    pltpu.sync_copy(u_hbm.at[pl.ds(n_pad - ZCH, ZCH)], rows_v.at[pl.ds(0, ZCH)])
    for z in range(stripe // ZCH):
        pltpu.sync_copy(rows_v.at[pl.ds(0, ZCH)],
                        acc_sp.at[pl.ds(s * stripe + z * ZCH, ZCH)])
    plsc.subcore_barrier()

    pltpu.async_copy(u_hbm.at[sidx_v.at[0]], rows_v, semA)

    def chunk(j2, carry):
        j = j2 * 2
        pltpu.make_async_copy(u_hbm.at[sidx_v.at[j]], rows_v, semA).wait()
        pltpu.async_copy(u_hbm.at[sidx_v.at[j + 1]], rows_b, semB)
        pltpu.sync_copy(rows_v, acc_sp.at[didx_v.at[j]], add=True)
        pltpu.make_async_copy(u_hbm.at[sidx_v.at[j + 1]], rows_b, semB).wait()

        @pl.when(j + 2 < n_ch)
        def _next():
            pltpu.async_copy(u_hbm.at[sidx_v.at[j + 2]], rows_v, semA)
        pltpu.sync_copy(rows_b, acc_sp.at[didx_v.at[j + 1]], add=True)
        return carry
    lax.fori_loop(0, n_ch // 2, chunk, 0)
    plsc.subcore_barrier()
    pltpu.sync_copy(acc_sp.at[pl.ds(s * stripe, stripe)],
                    out_hbm.at[c, pl.ds(s * stripe, stripe)])


def _make_sc_scatter(n_pad, w, n_ch):
    mesh = plsc.VectorSubcoreMesh(core_axis_name="c", subcore_axis_name="s")
    return pl.kernel(
        functools.partial(_sc_scatter_body, n_pad, w, n_ch),
        out_type=jax.ShapeDtypeStruct((2, n_pad, w), jnp.float32),
        mesh=mesh,
        compiler_params=pltpu.CompilerParams(use_tc_tiling_on_sc=False),
        scratch_types=[
            pltpu.VMEM((n_ch, CHUNK), jnp.int32),
            pltpu.VMEM((n_ch, CHUNK), jnp.int32),
            pltpu.VMEM((CHUNK, w), jnp.float32),
            pltpu.VMEM((CHUNK, w), jnp.float32),
            pltpu.VMEM_SHARED((n_pad, w), jnp.float32),
            pltpu.SemaphoreType.DMA,
            pltpu.SemaphoreType.DMA,
        ],
    )


def _mm_body(a_ref, w_ref, o_ref):
    o_ref[...] = jnp.dot(a_ref[...], w_ref[...], preferred_element_type=jnp.float32)


def _matmul(a, w, bn=512):
    n, k = a.shape
    m = w.shape[1]
    return pl.pallas_call(
        _mm_body,
        grid=(n // bn,),
        in_specs=[pl.BlockSpec((bn, k), lambda i: (i, 0)),
                  pl.BlockSpec((k, m), lambda i: (0, 0))],
        out_specs=pl.BlockSpec((bn, m), lambda i: (i, 0)),
        out_shape=jax.ShapeDtypeStruct((n, m), jnp.float32),
    )(a, w)


def _scale_body(hw_ref, c0_ref, c1_ref, act_ref, u_ref, a_ref):
    cnt = c0_ref[:, :1] + c1_ref[:, :1]
    act = act_ref[...]
    a = act / jnp.sqrt(cnt + 1.0)
    a_ref[...] = a
    u_ref[...] = hw_ref[...] * a


def _scale_table(hw_ext, cnt0, cnt1, act, bn=512):
    n, w = hw_ext.shape
    return pl.pallas_call(
        _scale_body,
        grid=(n // bn,),
        in_specs=[pl.BlockSpec((bn, w), lambda i: (i, 0)),
                  pl.BlockSpec((bn, 16), lambda i: (i, 0)),
                  pl.BlockSpec((bn, 16), lambda i: (i, 0)),
                  pl.BlockSpec((bn, 1), lambda i: (i, 0))],
        out_specs=[pl.BlockSpec((bn, w), lambda i: (i, 0)),
                   pl.BlockSpec((bn, 1), lambda i: (i, 0))],
        out_shape=[jax.ShapeDtypeStruct((n, w), jnp.float32),
                   jax.ShapeDtypeStruct((n, 1), jnp.float32)],
    )(hw_ext, cnt0, cnt1, act)


def _post_body(p0_ref, p1_ref, hw_ref, a_ref, act_ref, bias_ref, ws_ref,
               h_ref, sw_ref, us_ref):
    a = a_ref[...]
    conv = a * (p0_ref[...] + p1_ref[...]) + hw_ref[...] * (a * a)
    res = (conv + bias_ref[...]) * act_ref[...]
    h_ref[...] = res
    sw = jnp.dot(res, ws_ref[...], preferred_element_type=jnp.float32)
    sw_ref[...] = sw
    us_ref[...] = jnp.broadcast_to(sw * a, us_ref.shape)


def _post_conv(p0, p1, hw, a, act, bias, ws_row, bn=512):
    n, d = hw.shape
    return pl.pallas_call(
        _post_body,
        grid=(n // bn,),
        in_specs=[pl.BlockSpec((bn, d), lambda i: (i, 0)),
                  pl.BlockSpec((bn, d), lambda i: (i, 0)),
                  pl.BlockSpec((bn, d), lambda i: (i, 0)),
                  pl.BlockSpec((bn, 1), lambda i: (i, 0)),
                  pl.BlockSpec((bn, 1), lambda i: (i, 0)),
                  pl.BlockSpec((1, d), lambda i: (0, 0)),
                  pl.BlockSpec((d, 1), lambda i: (0, 0))],
        out_specs=[pl.BlockSpec((bn, d), lambda i: (i, 0)),
                   pl.BlockSpec((bn, 1), lambda i: (i, 0)),
                   pl.BlockSpec((bn, 16), lambda i: (i, 0))],
        out_shape=[jax.ShapeDtypeStruct((n, d), jnp.float32),
                   jax.ShapeDtypeStruct((n, 1), jnp.float32),
                   jax.ShapeDtypeStruct((n, 16), jnp.float32)],
    )(p0, p1, hw, a, act, bias, ws_row)


def _score_body(q0_ref, q1_ref, sw_ref, a_ref, act_ref, bs_ref, s_ref):
    a = a_ref[...]
    accs = q0_ref[:, :1] + q1_ref[:, :1]
    s_ref[...] = (a * accs + sw_ref[...] * (a * a) + bs_ref[...]) * act_ref[...]


def _score_post(q0, q1, sw, a, act, bs, bn=512):
    n = sw.shape[0]
    return pl.pallas_call(
        _score_body,
        grid=(n // bn,),
        in_specs=[pl.BlockSpec((bn, 16), lambda i: (i, 0)),
                  pl.BlockSpec((bn, 16), lambda i: (i, 0)),
                  pl.BlockSpec((bn, 1), lambda i: (i, 0)),
                  pl.BlockSpec((bn, 1), lambda i: (i, 0)),
                  pl.BlockSpec((bn, 1), lambda i: (i, 0)),
                  pl.BlockSpec((1, 1), lambda i: (0, 0))],
        out_specs=pl.BlockSpec((bn, 1), lambda i: (i, 0)),
        out_shape=jax.ShapeDtypeStruct((n, 1), jnp.float32),
    )(q0, q1, sw, a, act, bs)


def _topk_body(jch, jblk, sc_ref, bc_ref, ac_ref, sr_ref, br_ref, ar_ref,
               ilo_ref, ihi_ref, jlo_ref, jhi_ref, keep_ref):
    p = pl.program_id(0)
    bi = sc_ref.shape[0]
    si = sc_ref[...]
    gi = bc_ref[...]
    ai = ac_ref[...]
    ii = (lax.broadcasted_iota(jnp.int32, (bi, 1), 0).astype(jnp.float32)
          + jnp.float32(bi) * p.astype(jnp.float32))
    my_lo = ilo_ref[p]
    my_hi = ihi_ref[p]

    def jstep(j, carry):
        rank, cnt = carry

        def live(_):
            sj = sr_ref[pl.ds(j, 1), :]
            gj = br_ref[pl.ds(j, 1), :]
            aj = ar_ref[pl.ds(j, 1), :]
            jj = (lax.broadcasted_iota(jnp.int32, (1, jblk), 1).astype(jnp.float32)
                  + jnp.float32(jblk) * j.astype(jnp.float32))
            same = (gi == gj) & (aj > 0.0)
            better = (sj > si) | ((sj == si) & (jj < ii))
            r = jnp.sum(jnp.where(same & better, 1.0, 0.0), axis=1, keepdims=True)
            c = jnp.sum(jnp.where(same, 1.0, 0.0), axis=1, keepdims=True)
            return rank + r, cnt + c

        ov = (my_lo <= jhi_ref[j]) & (jlo_ref[j] <= my_hi)
        return lax.cond(ov, live, lambda _: (rank, cnt), 0)

    rank, cnt = lax.fori_loop(0, jch, jstep, (jnp.zeros((bi, 1), jnp.float32),
                                              jnp.zeros((bi, 1), jnp.float32)))
    k = jnp.ceil(RATIO * cnt)
    keep = jnp.where((ai > 0.0) & (rank < k), 1.0, 0.0)
    keep_ref[...] = jnp.broadcast_to(keep, keep_ref.shape)


def _topk(score_c, batch_c, act_c, score_r, batch_r, act_r, ilo, ihi, jlo, jhi,
          bi=128, jblk=512):
    n = score_c.shape[0]
    jch = n // jblk
    return pl.pallas_call(
        functools.partial(_topk_body, jch, jblk),
        grid=(n // bi,),
        in_specs=[pl.BlockSpec((bi, 1), lambda i: (i, 0)),
                  pl.BlockSpec((bi, 1), lambda i: (i, 0)),
                  pl.BlockSpec((bi, 1), lambda i: (i, 0)),
                  pl.BlockSpec((jch, jblk), lambda i: (0, 0)),
                  pl.BlockSpec((jch, jblk), lambda i: (0, 0)),
                  pl.BlockSpec((jch, jblk), lambda i: (0, 0)),
                  pl.BlockSpec(memory_space=pltpu.SMEM),
                  pl.BlockSpec(memory_space=pltpu.SMEM),
                  pl.BlockSpec(memory_space=pltpu.SMEM),
                  pl.BlockSpec(memory_space=pltpu.SMEM)],
        out_specs=pl.BlockSpec((bi, 16), lambda i: (i, 0)),
        out_shape=jax.ShapeDtypeStruct((n, 16), jnp.float32),
    )(score_c, batch_c, act_c, score_r, batch_r, act_r, ilo, ihi, jlo, jhi)


def _stats_body(h_ref, s_ref, m_ref, sum_ref, sq_ref, c_ref):
    @pl.when(pl.program_id(0) == 0)
    def _init():
        sum_ref[...] = jnp.zeros_like(sum_ref)
        sq_ref[...] = jnp.zeros_like(sq_ref)
        c_ref[...] = jnp.zeros_like(c_ref)
    m = m_ref[...]
    hm = h_ref[...] * jnp.tanh(s_ref[...]) * m
    sum_ref[...] += jnp.sum(hm, axis=0, keepdims=True)
    sq_ref[...] += jnp.sum(hm * hm, axis=0, keepdims=True)
    c_ref[...] += jnp.sum(m, axis=0, keepdims=True)


def _bn_stats(h, score, m, bn=512):
    n, d = h.shape
    return pl.pallas_call(
        _stats_body,
        grid=(n // bn,),
        in_specs=[pl.BlockSpec((bn, d), lambda i: (i, 0)),
                  pl.BlockSpec((bn, 1), lambda i: (i, 0)),
                  pl.BlockSpec((bn, 1), lambda i: (i, 0))],
        out_specs=[pl.BlockSpec((1, d), lambda i: (0, 0)),
                   pl.BlockSpec((1, d), lambda i: (0, 0)),
                   pl.BlockSpec((1, 1), lambda i: (0, 0))],
        out_shape=[jax.ShapeDtypeStruct((1, d), jnp.float32),
                   jax.ShapeDtypeStruct((1, d), jnp.float32),
                   jax.ShapeDtypeStruct((1, 1), jnp.float32)],
    )(h, score, m)


def _bnorm_body(with_mm, h_ref, s_ref, m_ref, sum_ref, sq_ref, c_ref,
                g_ref, be_ref, w_ref, h_out, hw_out):
    cnt = jnp.maximum(c_ref[0, 0], 1.0)
    mean = sum_ref[...] / cnt
    var = sq_ref[...] / cnt - mean * mean
    inv = lax.rsqrt(var + EPS)
    m = m_ref[...]
    hm = h_ref[...] * jnp.tanh(s_ref[...]) * m
    hn = ((hm - mean) * inv * g_ref[...] + be_ref[...]) * m
    hn = jnp.maximum(hn, 0.0)
    h_out[...] = hn
    if with_mm:
        hw_out[...] = jnp.dot(hn, w_ref[...], preferred_element_type=jnp.float32)


def _bn_apply(h, score, m, ssum, ssq, scnt, g, be, wcat=None, bn=512):
    n, d = h.shape
    with_mm = wcat is not None
    in_specs = [pl.BlockSpec((bn, d), lambda i: (i, 0)),
                pl.BlockSpec((bn, 1), lambda i: (i, 0)),
                pl.BlockSpec((bn, 1), lambda i: (i, 0)),
                pl.BlockSpec((1, d), lambda i: (0, 0)),
                pl.BlockSpec((1, d), lambda i: (0, 0)),
                pl.BlockSpec((1, 1), lambda i: (0, 0)),
                pl.BlockSpec((1, d), lambda i: (0, 0)),
                pl.BlockSpec((1, d), lambda i: (0, 0))]
    out_specs = [pl.BlockSpec((bn, d), lambda i: (i, 0))]
    out_shape = [jax.ShapeDtypeStruct((n, d), jnp.float32)]
    args = [h, score, m, ssum, ssq, scnt, g[None, :], be[None, :]]
    if with_mm:
        w = wcat.shape[1]
        in_specs.append(pl.BlockSpec((d, w), lambda i: (0, 0)))
        out_specs.append(pl.BlockSpec((bn, w), lambda i: (i, 0)))
        out_shape.append(jax.ShapeDtypeStruct((n, w), jnp.float32))
        args.append(wcat)
    else:
        in_specs.append(pl.BlockSpec((d, 8), lambda i: (0, 0)))
        args.append(jnp.zeros((d, 8), jnp.float32))
    body = functools.partial(_bnorm_body, with_mm)
    if not with_mm:
        def body(h_ref, s_ref, m_ref, sum_ref, sq_ref, c_ref, g_ref, be_ref,
                 w_ref, h_out):
            _bnorm_body(False, h_ref, s_ref, m_ref, sum_ref, sq_ref, c_ref,
                        g_ref, be_ref, w_ref, h_out, None)
    res = pl.pallas_call(
        body,
        grid=(n // bn,),
        in_specs=in_specs,
        out_specs=out_specs,
        out_shape=out_shape,
    )(*args)
    return res if with_mm else (res[0], None)


def _combine_body(ng, x_ref, h1_ref, h2_ref, b_ref, lw0_ref, lw1_ref, lw2_ref,
                  lb_ref, o_ref, p0_ref, p1_ref, p2_ref):
    @pl.when(pl.program_id(0) == 0)
    def _init():
        p0_ref[...] = jnp.zeros_like(p0_ref)
        p1_ref[...] = jnp.zeros_like(p1_ref)
        p2_ref[...] = jnp.zeros_like(p2_ref)
    hp = jax.lax.Precision.HIGHEST
    gc = lax.broadcasted_iota(jnp.int32, (ng, 1), 0).astype(jnp.float32)
    pt = jnp.where(b_ref[0] == gc, 1.0, 0.0)
    p0_ref[...] += jnp.dot(pt, x_ref[...], preferred_element_type=jnp.float32,
                           precision=hp)
    p1_ref[...] += jnp.dot(pt, h1_ref[...], preferred_element_type=jnp.float32,
                           precision=hp)
    p2_ref[...] += jnp.dot(pt, h2_ref[...], preferred_element_type=jnp.float32,
                           precision=hp)

    @pl.when(pl.program_id(0) == pl.num_programs(0) - 1)
    def _final():
        out = jnp.dot(p0_ref[...], lw0_ref[...],
                      preferred_element_type=jnp.float32)
        out += jnp.dot(p1_ref[...], lw1_ref[...],
                       preferred_element_type=jnp.float32)
        out += jnp.dot(p2_ref[...], lw2_ref[...],
                       preferred_element_type=jnp.float32)
        o_ref[...] = out + lb_ref[...]


def _combine(x, h1, h2, batch_row, lw0, lw1, lw2, lbsum, ng, nc, bn=512):
    n, d = x.shape
    return pl.pallas_call(
        functools.partial(_combine_body, ng),
        grid=(n // bn,),
        in_specs=[pl.BlockSpec((bn, d), lambda i: (i, 0)),
                  pl.BlockSpec((bn, d), lambda i: (i, 0)),
                  pl.BlockSpec((bn, d), lambda i: (i, 0)),
                  pl.BlockSpec((1, 1, bn), lambda i: (i, 0, 0)),
                  pl.BlockSpec((d, nc), lambda i: (0, 0)),
                  pl.BlockSpec((d, nc), lambda i: (0, 0)),
                  pl.BlockSpec((d, nc), lambda i: (0, 0)),
                  pl.BlockSpec((1, nc), lambda i: (0, 0))],
        out_specs=pl.BlockSpec((ng, nc), lambda i: (0, 0)),
        out_shape=jax.ShapeDtypeStruct((ng, nc), jnp.float32),
        scratch_shapes=[pltpu.VMEM((ng, d), jnp.float32)] * 3,
    )(x, h1, h2, batch_row.reshape(n // bn, 1, bn), lw0, lw1, lw2, lbsum)


def kernel(x, edge_index, batch, W0, b0, Ws0, bs0, g0, be0, W1, b1, Ws1, bs1,
           g1, be1, LW0, Lb0, LW1, Lb1, LW2, Lb2):
    n, d = x.shape
    e = edge_index.shape[1]
    nc = LW0.shape[1]
    ng = 64
    n_pad = _round_up(n + ZCH, 2048)
    e_pad = _round_up(e, NW * CHUNK * 2)
    n_ch = e_pad // (NW * CHUNK)
    n_trash = n_pad - n

    xp = jnp.pad(x, ((0, n_pad - n), (0, 0)))
    batch_p = jnp.pad(batch, (0, n_pad - n), constant_values=ng)
    bf = batch_p.astype(jnp.float32)
    batch_c = bf[:, None]
    batch_r = bf.reshape(-1, 512)
    bi_lo = batch_p.reshape(-1, 128)[:, 0]
    bi_hi = batch_p.reshape(-1, 128)[:, 127]
    jc_lo = batch_p.reshape(-1, 512)[:, 0]
    jc_hi = batch_p.reshape(-1, 512)[:, 511]

    npad_e = e_pad - e
    trash = n + (jnp.arange(npad_e, dtype=jnp.int32) % n_trash)
    src3 = jnp.concatenate([edge_index[0], trash]).reshape(NW, n_ch, CHUNK)
    dst3 = jnp.concatenate([edge_index[1], trash]).reshape(NW, n_ch, CHUNK)

    act = jnp.pad(jnp.ones((n, 1), jnp.float32), ((0, n_pad - n), (0, 0)))
    act_r = act.reshape(-1, 512)
    m_tbl = jnp.pad(jnp.ones((n, 16), jnp.float32), ((0, n_pad - n), (0, 0)))

    sc_cnt = _make_sc_scatter(n_pad, 16, n_ch)
    sc_conv = _make_sc_scatter(n_pad, d, n_ch)

    convW = [(b0[None, :], Ws0.reshape(d, 1), bs0.reshape(1, 1)),
             (b1[None, :], Ws1.reshape(d, 1), bs1.reshape(1, 1))]
    bnW = [(g0, be0), (g1, be1)]
    hidden = []
    hw = _matmul(xp, W0)
    for l in range(2):
        bias, ws_row, bs = convW[l]
        cntp = sc_cnt(m_tbl, src3, dst3)
        u, a = _scale_table(hw, cntp[0], cntp[1], act)
        accp = sc_conv(u, src3, dst3)
        hcv, sw, us16 = _post_conv(accp[0], accp[1], hw, a, act, bias, ws_row)
        qp = sc_cnt(us16, src3, dst3)
        score = _score_post(qp[0], qp[1], sw, a, act, bs)
        sr = score.reshape(-1, 512)
        keep16 = _topk(score, batch_c, act, sr, batch_r, act_r,
                       bi_lo, bi_hi, jc_lo, jc_hi)
        m = keep16[:, :1]
        ssum, ssq, scnt = _bn_stats(hcv, score, m)
        g, be = bnW[l]
        if l == 0:
            h, hw = _bn_apply(hcv, score, m, ssum, ssq, scnt, g, be, W1)
        else:
            h, _ = _bn_apply(hcv, score, m, ssum, ssq, scnt, g, be, None)
        hidden.append(h)
        act = m
        act_r = act.reshape(-1, 512)
        m_tbl = keep16

    lbsum = (Lb0 + Lb1 + Lb2)[None, :]
    out = _combine(xp, hidden[0], hidden[1], batch_r, LW0, LW1, LW2, lbsum,
                   ng, nc)
    return out
